# baseline (speedup 1.0000x reference)
import functools

import jax
import jax.numpy as jnp
from jax import lax
from jax.experimental import pallas as pl
from jax.experimental.pallas import tpu as pltpu
from jax.experimental.pallas import tpu_sc as plsc

Z = 64
B = 16384
V = 2 ** 20
NC = 2
NS = 16
NW = NC * NS
BPW = B // NW

_mesh = plsc.VectorSubcoreMesh(core_axis_name="c", subcore_axis_name="s")

TB = 512


def _bf16_bits(x):
    return lax.bitcast_convert_type(
        x.astype(jnp.bfloat16).astype(jnp.float32), jnp.uint32
    )


def _tr_body(src_ref, dst_ref):
    t = lax.transpose(src_ref[...], (1, 0))
    lo = _bf16_bits(t[:, :Z // 2]) >> 16
    hi = _bf16_bits(t[:, Z // 2:])
    dst_ref[...] = (lo | hi).astype(jnp.int32)


def _transpose_table(embT):
    return pl.pallas_call(
        _tr_body,
        grid=(V // TB,),
        in_specs=[pl.BlockSpec((Z, TB), lambda i: (0, i))],
        out_specs=pl.BlockSpec((TB, Z // 2), lambda i: (i, 0)),
        out_shape=jax.ShapeDtypeStruct((V, Z // 2), jnp.int32),
    )(embT)


@functools.partial(
    pl.kernel,
    mesh=_mesh,
    out_type=jax.ShapeDtypeStruct((B, Z // 2), jnp.int32),
    scratch_types=[
        pltpu.VMEM((BPW,), jnp.int32),
        pltpu.VMEM((BPW, Z // 2), jnp.int32),
        pltpu.SemaphoreType.DMA,
    ],
)
def _sc_gather(idx_hbm, table_hbm, out_hbm, idx_v, rows_v, sem):
    wid = lax.axis_index("s") * NC + lax.axis_index("c")
    pltpu.sync_copy(idx_hbm.at[wid], idx_v)

    def body(g, carry):
        vec = idx_v[pl.ds(g * 16, 16)]
        for l in range(16):
            r = vec[l]
            pltpu.async_copy(
                table_hbm.at[pl.ds(r, 1)],
                rows_v.at[pl.ds(g * 16 + l, 1)],
                sem,
            )
        return carry

    lax.fori_loop(0, BPW // 16, body, None)
    pltpu.make_async_copy(table_hbm.at[pl.ds(0, BPW)], rows_v, sem).wait()
    pltpu.sync_copy(rows_v, out_hbm.at[pl.ds(wid * BPW, BPW)])


BB = 2048


def _mlp_body(g_ref, w1a_ref, w1b_ref, b1_ref, wmu_ref, bmu_ref, wlv_ref,
              blv_ref, mu_ref, lv_ref):
    w = lax.bitcast_convert_type(g_ref[...], jnp.uint32)
    ga = lax.bitcast_convert_type(w << 16, jnp.float32)
    gb = lax.bitcast_convert_type(w & jnp.uint32(0xFFFF0000), jnp.float32)
    dn = (((1,), (1,)), ((), ()))
    h = lax.dot_general(ga, w1a_ref[...], dn,
                        preferred_element_type=jnp.float32,
                        precision=lax.Precision.HIGHEST)
    h = h + lax.dot_general(gb, w1b_ref[...], dn,
                            preferred_element_type=jnp.float32,
                            precision=lax.Precision.HIGHEST)
    h = h + b1_ref[...]
    h = jnp.where(h >= 0, h, 0.01 * h)
    mu_ref[...] = lax.dot_general(h, wmu_ref[...], dn,
                                  preferred_element_type=jnp.float32,
                                  precision=lax.Precision.HIGHEST) + bmu_ref[...]
    lv_ref[...] = lax.dot_general(h, wlv_ref[...], dn,
                                  preferred_element_type=jnp.float32,
                                  precision=lax.Precision.HIGHEST) + blv_ref[...]


def _mlp(g, W1, b1, Wmu, bmu, Wlv, blv):
    hspec = pl.BlockSpec((Z, Z // 2), lambda i: (0, 0))
    wspec = pl.BlockSpec((Z, Z), lambda i: (0, 0))
    bspec = pl.BlockSpec((1, Z), lambda i: (0, 0))
    return pl.pallas_call(
        _mlp_body,
        grid=(B // BB,),
        in_specs=[
            pl.BlockSpec((BB, Z // 2), lambda i: (i, 0)),
            hspec, hspec, bspec, wspec, bspec, wspec, bspec,
        ],
        out_specs=[
            pl.BlockSpec((BB, Z), lambda i: (i, 0)),
            pl.BlockSpec((BB, Z), lambda i: (i, 0)),
        ],
        out_shape=[
            jax.ShapeDtypeStruct((B, Z), jnp.float32),
            jax.ShapeDtypeStruct((B, Z), jnp.float32),
        ],
    )(g, W1[:, :Z // 2], W1[:, Z // 2:], b1.reshape(1, Z),
      Wmu, bmu.reshape(1, Z), Wlv, blv.reshape(1, Z))


def kernel(x, emb, W1, b1, Wmu, bmu, Wlv, blv):
    xr = x.astype(jnp.int32).reshape(NW, BPW)
    table = _transpose_table(emb.T)
    g = _sc_gather(xr, table)
    mu, lv = _mlp(g, W1, b1, Wmu, bmu, Wlv, blv)
    return (mu, lv)


# speedup vs baseline: 2.2442x; 2.2442x over previous
import functools

import jax
import jax.numpy as jnp
from jax import lax
from jax.experimental import pallas as pl
from jax.experimental.pallas import tpu as pltpu
from jax.experimental.pallas import tpu_sc as plsc

Z = 64
B = 16384
V = 2 ** 20
NC = 2
NS = 16
NW = NC * NS
BPW = B // NW

_mesh = plsc.VectorSubcoreMesh(core_axis_name="c", subcore_axis_name="s")

TB = 2048


def _bf16_bits(x):
    return lax.bitcast_convert_type(
        x.astype(jnp.bfloat16).astype(jnp.float32), jnp.uint32
    )


def _tr_body(src_ref, dst_ref):
    src = src_ref[...]
    ii = lax.broadcasted_iota(jnp.int32, (Z, Z), 0)
    jj = lax.broadcasted_iota(jnp.int32, (Z, Z), 1)
    eye = (ii == jj).astype(jnp.float32)
    t = lax.dot_general(src, eye, (((0,), (0,)), ((), ())),
                        preferred_element_type=jnp.float32)
    lo = _bf16_bits(t[:, :Z // 2]) >> 16
    hi = _bf16_bits(t[:, Z // 2:])
    dst_ref[...] = (lo | hi).astype(jnp.int32)


def _transpose_table(embT):
    return pl.pallas_call(
        _tr_body,
        grid=(V // TB,),
        in_specs=[pl.BlockSpec((Z, TB), lambda i: (0, i))],
        out_specs=pl.BlockSpec((TB, Z // 2), lambda i: (i, 0)),
        out_shape=jax.ShapeDtypeStruct((V, Z // 2), jnp.int32),
    )(embT)


@functools.partial(
    pl.kernel,
    mesh=_mesh,
    out_type=jax.ShapeDtypeStruct((B, Z // 2), jnp.int32),
    scratch_types=[
        pltpu.VMEM((BPW,), jnp.int32),
        pltpu.VMEM((BPW, Z // 2), jnp.int32),
        pltpu.SemaphoreType.DMA,
    ],
)
def _sc_gather(idx_hbm, table_hbm, out_hbm, idx_v, rows_v, sem):
    wid = lax.axis_index("s") * NC + lax.axis_index("c")
    pltpu.sync_copy(idx_hbm.at[wid], idx_v)

    def body(g, carry):
        vec = idx_v[pl.ds(g * 16, 16)]
        for l in range(16):
            r = vec[l]
            pltpu.async_copy(
                table_hbm.at[pl.ds(r, 1)],
                rows_v.at[pl.ds(g * 16 + l, 1)],
                sem,
            )
        return carry

    lax.fori_loop(0, BPW // 16, body, None)
    pltpu.make_async_copy(table_hbm.at[pl.ds(0, BPW)], rows_v, sem).wait()
    pltpu.sync_copy(rows_v, out_hbm.at[pl.ds(wid * BPW, BPW)])


BB = 2048


def _mlp_body(g_ref, w1a_ref, w1b_ref, b1_ref, wmu_ref, bmu_ref, wlv_ref,
              blv_ref, mu_ref, lv_ref):
    w = lax.bitcast_convert_type(g_ref[...], jnp.uint32)
    ga = lax.bitcast_convert_type(w << 16, jnp.float32)
    gb = lax.bitcast_convert_type(w & jnp.uint32(0xFFFF0000), jnp.float32)
    dn = (((1,), (1,)), ((), ()))
    h = lax.dot_general(ga, w1a_ref[...], dn,
                        preferred_element_type=jnp.float32,
                        precision=lax.Precision.HIGHEST)
    h = h + lax.dot_general(gb, w1b_ref[...], dn,
                            preferred_element_type=jnp.float32,
                            precision=lax.Precision.HIGHEST)
    h = h + b1_ref[...]
    h = jnp.where(h >= 0, h, 0.01 * h)
    mu_ref[...] = lax.dot_general(h, wmu_ref[...], dn,
                                  preferred_element_type=jnp.float32,
                                  precision=lax.Precision.HIGHEST) + bmu_ref[...]
    lv_ref[...] = lax.dot_general(h, wlv_ref[...], dn,
                                  preferred_element_type=jnp.float32,
                                  precision=lax.Precision.HIGHEST) + blv_ref[...]


def _mlp(g, W1, b1, Wmu, bmu, Wlv, blv):
    hspec = pl.BlockSpec((Z, Z // 2), lambda i: (0, 0))
    wspec = pl.BlockSpec((Z, Z), lambda i: (0, 0))
    bspec = pl.BlockSpec((1, Z), lambda i: (0, 0))
    return pl.pallas_call(
        _mlp_body,
        grid=(B // BB,),
        in_specs=[
            pl.BlockSpec((BB, Z // 2), lambda i: (i, 0)),
            hspec, hspec, bspec, wspec, bspec, wspec, bspec,
        ],
        out_specs=[
            pl.BlockSpec((BB, Z), lambda i: (i, 0)),
            pl.BlockSpec((BB, Z), lambda i: (i, 0)),
        ],
        out_shape=[
            jax.ShapeDtypeStruct((B, Z), jnp.float32),
            jax.ShapeDtypeStruct((B, Z), jnp.float32),
        ],
    )(g, W1[:, :Z // 2], W1[:, Z // 2:], b1.reshape(1, Z),
      Wmu, bmu.reshape(1, Z), Wlv, blv.reshape(1, Z))


def kernel(x, emb, W1, b1, Wmu, bmu, Wlv, blv):
    xr = x.astype(jnp.int32).reshape(NW, BPW)
    table = _transpose_table(emb.T)
    g = _sc_gather(xr, table)
    mu, lv = _mlp(g, W1, b1, Wmu, bmu, Wlv, blv)
    return (mu, lv)
